# Initial kernel scaffold; baseline (speedup 1.0000x reference)
import jax
import jax.numpy as jnp
from jax.experimental import pallas as pl


def kernel(bond_fea, angle_fea, species, nbr_idx, crys_idx, params):
    raise NotImplementedError("write your pallas kernel here")


# speedup vs baseline: 3.6649x; 3.6649x over previous
import functools

import numpy as np
import jax
import jax.numpy as jnp
from jax import lax
from jax.experimental import pallas as pl
from jax.experimental.pallas import tpu as pltpu
from jax.experimental.pallas import tpu_sc as plsc

_N = 10000
_NEIGH = 12
_NPAD = 12288
_CHUNK = 384
_NSUB = 3
_ZROWS = _NPAD // 16
_R = 512
_NBLK = _NPAD // _R
_W = 1536
_NPAIR = 6

_INV_GB2 = float((64.0 / 8.0) ** 2)
_INV_GA2 = float((4.0 / 2.0) ** 2)

_BF = jnp.bfloat16


def _dot(a, b):
    return jnp.dot(a.astype(_BF), b, preferred_element_type=jnp.float32)


def _k1_body(bond_ref, ang_ref, fb_ref, fa_ref, x_ref):
    pid = pl.program_id(0)
    bond = bond_ref[...]
    fb = fb_ref[0][None, :]
    pieces = [jnp.exp(-((bond[:, k:k + 1] - fb) ** 2) * _INV_GB2)
              for k in range(_NEIGH)]
    ang = ang_ref[...]
    fa = fa_ref[...]
    for j in range(4):
        pieces.append(jnp.exp(-((ang - fa[0, j]) ** 2) * _INV_GA2))
    pieces.append(jnp.zeros((_R, _W - 1344), jnp.float32))
    x = jnp.concatenate(pieces, axis=1)
    rows = pid * _R + lax.broadcasted_iota(jnp.int32, (_R, 1), 0)
    x_ref[...] = jnp.where(rows < _N, x, 0.0)


def _k1(bond_p, ang_p, fb, fa):
    return pl.pallas_call(
        _k1_body,
        grid=(_NBLK,),
        in_specs=[
            pl.BlockSpec((_R, 12), lambda i: (i, 0)),
            pl.BlockSpec((_R, 144), lambda i: (i, 0)),
            pl.BlockSpec((8, 64), lambda i: (0, 0)),
            pl.BlockSpec((8, 4), lambda i: (0, 0)),
        ],
        out_specs=pl.BlockSpec((_R, _W), lambda i: (i, 0)),
        out_shape=jax.ShapeDtypeStruct((_NPAD, _W), jnp.float32),
    )(bond_p, ang_p, fb, fa)


def _sc_scatter_wide(x, idx_h, zs):
    mesh = plsc.VectorSubcoreMesh(core_axis_name="c", subcore_axis_name="s")

    @functools.partial(
        pl.kernel,
        mesh=mesh,
        out_type=jax.ShapeDtypeStruct((_NPAD, _W), jnp.float32),
        compiler_params=pltpu.CompilerParams(use_tc_tiling_on_sc=False),
        scratch_types=[
            pltpu.VMEM((128, 128), jnp.float32),
            pltpu.VMEM((_NEIGH, 6, 128), jnp.int32),
            pltpu.VMEM_SHARED((_NPAD, 128), jnp.float32),
        ],
    )
    def k(x_hbm, idx_hbm, zs_hbm, out_hbm, x_v, idx_v, acc):
        cid = lax.axis_index("c")
        sid = lax.axis_index("s")
        zbase = sid * _ZROWS
        pltpu.sync_copy(idx_hbm.at[sid], idx_v)
        for pp in range(_NPAIR):
            colc = (pp * 2 + cid) * 128
            pltpu.sync_copy(zs_hbm, x_v)
            for z in range(_ZROWS // 128):
                pltpu.sync_copy(x_v, acc.at[pl.ds(zbase + z * 128, 128)])
            plsc.subcore_barrier()
            for j in range(6):
                pltpu.sync_copy(
                    x_hbm.at[pl.ds(zbase + j * 128, 128),
                             pl.ds(colc, 128)], x_v)

                def body(kk, carry, _j=j):
                    pltpu.sync_copy(x_v, acc.at[idx_v.at[kk, _j]], add=True)
                    return carry

                lax.fori_loop(0, _NEIGH, body, 0)
            plsc.subcore_barrier()
            for z in range(_ZROWS // 128):
                pltpu.sync_copy(acc.at[pl.ds(zbase + z * 128, 128)], x_v)
                pltpu.sync_copy(
                    x_v, out_hbm.at[pl.ds(zbase + z * 128, 128),
                                    pl.ds(colc, 128)])
            plsc.subcore_barrier()

    return k(x, idx_h, zs)


def _sc_scatter64(y, idx_g, zs):
    mesh = plsc.VectorSubcoreMesh(core_axis_name="c", subcore_axis_name="s")

    @functools.partial(
        pl.kernel,
        mesh=mesh,
        out_type=jax.ShapeDtypeStruct((2 * _NPAD, 64), jnp.float32),
        compiler_params=pltpu.CompilerParams(use_tc_tiling_on_sc=False),
        scratch_types=[
            pltpu.VMEM((_CHUNK, 64), jnp.float32),
            pltpu.VMEM((_NEIGH, _NSUB, 128), jnp.int32),
            pltpu.VMEM((128, 64), jnp.float32),
            pltpu.VMEM_SHARED((_NPAD, 64), jnp.float32),
        ],
    )
    def k(y_hbm, idx_hbm, zs_hbm, out_hbm, y_v, idx_v, st_v, acc):
        cid = lax.axis_index("c")
        sid = lax.axis_index("s")
        wid = sid * 2 + cid
        base = wid * _CHUNK
        zbase = sid * _ZROWS
        pltpu.sync_copy(zs_hbm, st_v)
        for z in range(_ZROWS // 128):
            pltpu.sync_copy(st_v, acc.at[pl.ds(zbase + z * 128, 128)])
        pltpu.sync_copy(y_hbm.at[pl.ds(base, _CHUNK)], y_v)
        pltpu.sync_copy(idx_hbm.at[wid], idx_v)
        plsc.subcore_barrier()

        def body(kk, carry):
            for sub in range(_NSUB):
                pltpu.sync_copy(y_v.at[pl.ds(sub * 128, 128)],
                                acc.at[idx_v.at[kk, sub]], add=True)
            return carry

        lax.fori_loop(0, _NEIGH, body, 0)
        plsc.subcore_barrier()
        for z in range(_ZROWS // 128):
            pltpu.sync_copy(acc.at[pl.ds(zbase + z * 128, 128)], st_v)
            pltpu.sync_copy(
                st_v, out_hbm.at[pl.ds(cid * _NPAD + zbase + z * 128, 128)])

    return k(y, idx_g, zs).reshape(2, _NPAD, 64)


def _tail1_body(x_ref, agg_ref, w1_ref, w2_ref, bias_ref, x1_ref, st_ref):
    pid = pl.program_id(0)
    h = x_ref[...] + agg_ref[...]
    y = _dot(h, w1_ref[...]) + bias_ref[0][None, :]
    h1 = jnp.maximum(y, 0.0)
    x1 = jnp.maximum(_dot(h1, w2_ref[...]) + bias_ref[1][None, :], 0.0)
    rows = pid * _R + lax.broadcasted_iota(jnp.int32, (_R, 1), 0)
    x1 = jnp.where(rows < _N, x1, 0.0)
    x1_ref[...] = x1

    @pl.when(pid == 0)
    def _():
        st_ref[...] = jnp.zeros_like(st_ref)

    st_ref[...] += jnp.concatenate(
        [jnp.sum(x1, axis=0, keepdims=True),
         jnp.sum(x1 * x1, axis=0, keepdims=True),
         jnp.zeros((6, 64), jnp.float32)], axis=0)


def _tail1(x, agg, w1, w2, bias):
    return pl.pallas_call(
        _tail1_body,
        grid=(_NBLK,),
        in_specs=[
            pl.BlockSpec((_R, _W), lambda i: (i, 0)),
            pl.BlockSpec((_R, _W), lambda i: (i, 0)),
            pl.BlockSpec((_W, 64), lambda i: (0, 0)),
            pl.BlockSpec((64, 64), lambda i: (0, 0)),
            pl.BlockSpec((8, 64), lambda i: (0, 0)),
        ],
        out_specs=[
            pl.BlockSpec((_R, 64), lambda i: (i, 0)),
            pl.BlockSpec((8, 64), lambda i: (0, 0)),
        ],
        out_shape=[
            jax.ShapeDtypeStruct((_NPAD, 64), jnp.float32),
            jax.ShapeDtypeStruct((8, 64), jnp.float32),
        ],
    )(x, agg, w1, w2, bias)


def _bn_body(x_ref, sc_ref, z_ref):
    pid = pl.program_id(0)
    z = ((x_ref[...] - sc_ref[0][None, :]) * sc_ref[1][None, :]
         * sc_ref[2][None, :] + sc_ref[3][None, :])
    rows = pid * _R + lax.broadcasted_iota(jnp.int32, (_R, 1), 0)
    z_ref[...] = jnp.where(rows < _N, z, 0.0)


def _bn(x, sc):
    return pl.pallas_call(
        _bn_body,
        grid=(_NBLK,),
        in_specs=[
            pl.BlockSpec((_R, 64), lambda i: (i, 0)),
            pl.BlockSpec((8, 64), lambda i: (0, 0)),
        ],
        out_specs=pl.BlockSpec((_R, 64), lambda i: (i, 0)),
        out_shape=jax.ShapeDtypeStruct((_NPAD, 64), jnp.float32),
    )(x, sc)


def _tail2_body(z_ref, agg_ref, w1_ref, w2_ref, bias_ref, x2_ref, st_ref):
    pid = pl.program_id(0)
    a = agg_ref[0] + agg_ref[1]
    h = z_ref[...] + a
    y = _dot(h, w1_ref[...]) + bias_ref[0][None, :]
    h1 = jnp.maximum(y, 0.0)
    x2 = jnp.maximum(_dot(h1, w2_ref[...]) + bias_ref[1][None, :], 0.0)
    rows = pid * _R + lax.broadcasted_iota(jnp.int32, (_R, 1), 0)
    x2 = jnp.where(rows < _N, x2, 0.0)
    x2_ref[...] = x2

    @pl.when(pid == 0)
    def _():
        st_ref[...] = jnp.zeros_like(st_ref)

    st_ref[...] += jnp.concatenate(
        [jnp.sum(x2, axis=0, keepdims=True),
         jnp.sum(x2 * x2, axis=0, keepdims=True),
         jnp.zeros((6, 64), jnp.float32)], axis=0)


def _tail2(z, agg, w1, w2, bias):
    return pl.pallas_call(
        _tail2_body,
        grid=(_NBLK,),
        in_specs=[
            pl.BlockSpec((_R, 64), lambda i: (i, 0)),
            pl.BlockSpec((2, _R, 64), lambda i: (0, i, 0)),
            pl.BlockSpec((64, 64), lambda i: (0, 0)),
            pl.BlockSpec((64, 64), lambda i: (0, 0)),
            pl.BlockSpec((8, 64), lambda i: (0, 0)),
        ],
        out_specs=[
            pl.BlockSpec((_R, 64), lambda i: (i, 0)),
            pl.BlockSpec((8, 64), lambda i: (0, 0)),
        ],
        out_shape=[
            jax.ShapeDtypeStruct((_NPAD, 64), jnp.float32),
            jax.ShapeDtypeStruct((8, 64), jnp.float32),
        ],
    )(z, agg, w1, w2, bias)


def _final_body(x_ref, sc_ref, f1_ref, f2_ref, fw_ref, fb_ref, o_ref):
    z = ((x_ref[...] - sc_ref[0][None, :]) * sc_ref[1][None, :]
         * sc_ref[2][None, :] + sc_ref[3][None, :])
    c1 = _dot(z, f1_ref[...]) + sc_ref[4][None, :32]
    c2 = _dot(c1, f2_ref[...]) + sc_ref[5][None, :16]
    o_ref[...] = _dot(c2, fw_ref[...]) + fb_ref[...]


def _final(x, sc, f1, f2, fw, fb):
    return pl.pallas_call(
        _final_body,
        grid=(_NBLK,),
        in_specs=[
            pl.BlockSpec((_R, 64), lambda i: (i, 0)),
            pl.BlockSpec((8, 64), lambda i: (0, 0)),
            pl.BlockSpec((64, 32), lambda i: (0, 0)),
            pl.BlockSpec((32, 16), lambda i: (0, 0)),
            pl.BlockSpec((16, 2), lambda i: (0, 0)),
            pl.BlockSpec((1, 2), lambda i: (0, 0)),
        ],
        out_specs=pl.BlockSpec((_R, 2), lambda i: (i, 0)),
        out_shape=jax.ShapeDtypeStruct((_NPAD, 2), jnp.float32),
    )(x, sc, f1, f2, fw, fb)


def _stats_to_scale(st, g, b):
    mu = st[0] / _N
    var = st[1] / _N - mu * mu
    inv = 1.0 / jnp.sqrt(var + 1e-5)
    return mu, inv, g, b


def _blockdiag(wb, wa, n1, n2):
    return (jnp.zeros((2 * n1, 2 * n2), jnp.float32)
            .at[:n1, :n2].set(wb).at[n1:, n2:].set(wa))


def kernel(bond_fea, angle_fea, species, nbr_idx, crys_idx, params):
    p = params
    f32 = jnp.float32
    bond_p = jnp.zeros((_NPAD, 12), f32).at[:_N].set(bond_fea)
    ang_p = jnp.zeros((_NPAD, 144), f32).at[:_N].set(
        angle_fea.reshape(_N, 144))
    nbr_p = jnp.zeros((_NPAD, _NEIGH), jnp.int32).at[:_N].set(
        nbr_idx.astype(jnp.int32))
    idx_h = (nbr_p.reshape(16, 6, 128, _NEIGH)
             .transpose(0, 3, 1, 2))
    idx_g = (nbr_p.reshape(32, _NSUB, 128, _NEIGH)
             .transpose(0, 3, 1, 2))
    zs128 = jnp.zeros((128, 128), f32)
    zs64 = jnp.zeros((128, 64), f32)
    fb = jnp.zeros((8, 64), f32).at[0].set(jnp.linspace(0.0, 8.0, 64))
    fa = jnp.zeros((8, 4), f32).at[0].set(jnp.linspace(-1.0, 1.0, 4))

    x = _k1(bond_p, ang_p, fb, fa)
    agg = _sc_scatter_wide(x, idx_h, zs128)

    w1a_perm = (p['nn1a_W1'].reshape(144, 4, 32)
                .transpose(1, 0, 2).reshape(576, 32))
    w1cat = (jnp.zeros((_W, 64), f32)
             .at[:768, :32].set(p['nn1b_W1'])
             .at[768:1344, 32:].set(w1a_perm)).astype(_BF)
    w2bd = _blockdiag(p['nn1b_W2'], p['nn1a_W2'], 32, 32).astype(_BF)
    bias1 = (jnp.zeros((8, 64), f32)
             .at[0].set(jnp.concatenate([p['nn1b_b1'], p['nn1a_b1']]))
             .at[1].set(jnp.concatenate([p['nn1b_b2'], p['nn1a_b2']])))
    x1, st1 = _tail1(x, agg, w1cat, w2bd, bias1)

    mu1, inv1, g1, b1 = _stats_to_scale(
        st1, jnp.concatenate([p['bn1b_g'], p['bn1a_g']]),
        jnp.concatenate([p['bn1b_b'], p['bn1a_b']]))
    sc1 = (jnp.zeros((8, 64), f32).at[0].set(mu1).at[1].set(inv1)
           .at[2].set(g1).at[3].set(b1))
    z1 = _bn(x1, sc1)

    agg2 = _sc_scatter64(z1, idx_g, zs64)

    w1bd2 = _blockdiag(p['nn2b_W1'], p['nn2a_W1'], 32, 32).astype(_BF)
    w2bd2 = _blockdiag(p['nn2b_W2'], p['nn2a_W2'], 32, 32).astype(_BF)
    bias2 = (jnp.zeros((8, 64), f32)
             .at[0].set(jnp.concatenate([p['nn2b_b1'], p['nn2a_b1']]))
             .at[1].set(jnp.concatenate([p['nn2b_b2'], p['nn2a_b2']])))
    x2, st2 = _tail2(z1, agg2, w1bd2, w2bd2, bias2)

    mu2, inv2, g2, b2 = _stats_to_scale(
        st2, jnp.concatenate([p['bn2b_g'], p['bn2a_g']]),
        jnp.concatenate([p['bn2b_b'], p['bn2a_b']]))
    sc2 = (jnp.zeros((8, 64), f32).at[0].set(mu2).at[1].set(inv2)
           .at[2].set(g2).at[3].set(b2)
           .at[4, :32].set(jnp.concatenate([p['fc1b_b'], p['fc1a_b']]))
           .at[5, :16].set(jnp.concatenate([p['fc2b_b'], p['fc2a_b']])))
    f1 = _blockdiag(p['fc1b_W'], p['fc1a_W'], 32, 16).astype(_BF)
    f2 = _blockdiag(p['fc2b_W'], p['fc2a_W'], 16, 8).astype(_BF)
    out = _final(x2, sc2, f1, f2, p['fc_W'].astype(_BF),
                 p['fc_b'][None, :])
    return out[:_N]


# speedup vs baseline: 3.7292x; 1.0175x over previous
import functools

import numpy as np
import jax
import jax.numpy as jnp
from jax import lax
from jax.experimental import pallas as pl
from jax.experimental.pallas import tpu as pltpu
from jax.experimental.pallas import tpu_sc as plsc

_N = 10000
_NEIGH = 12
_NPAD = 12288
_CHUNK = 384
_NSUB = 3
_ZROWS = _NPAD // 16
_R = 512
_NBLK = _NPAD // _R
_W = 1408
_NPAN = 11
_NPAIR = 6

_INV_GB2 = float((64.0 / 8.0) ** 2)
_INV_GA2 = float((4.0 / 2.0) ** 2)

_BF = jnp.bfloat16


def _dot(a, b):
    return jnp.dot(a.astype(_BF), b, preferred_element_type=jnp.float32)


def _k1_body(bond_ref, ang_ref, fb_ref, fa_ref, x_ref):
    pid = pl.program_id(0)
    bond = bond_ref[...]
    fb = fb_ref[0][None, :]
    pieces = [jnp.exp(-((bond[:, k:k + 1] - fb) ** 2) * _INV_GB2)
              for k in range(_NEIGH)]
    ang = ang_ref[...]
    fa = fa_ref[...]
    for j in range(4):
        pieces.append(jnp.exp(-((ang - fa[0, j]) ** 2) * _INV_GA2))
    pieces.append(jnp.zeros((_R, _W - 1344), jnp.float32))
    x = jnp.concatenate(pieces, axis=1)
    rows = pid * _R + lax.broadcasted_iota(jnp.int32, (_R, 1), 0)
    x_ref[...] = jnp.where(rows < _N, x, 0.0)


def _k1(bond_p, ang_p, fb, fa):
    return pl.pallas_call(
        _k1_body,
        grid=(_NBLK,),
        in_specs=[
            pl.BlockSpec((_R, 12), lambda i: (i, 0)),
            pl.BlockSpec((_R, 144), lambda i: (i, 0)),
            pl.BlockSpec((8, 64), lambda i: (0, 0)),
            pl.BlockSpec((8, 4), lambda i: (0, 0)),
        ],
        out_specs=pl.BlockSpec((_R, _W), lambda i: (i, 0)),
        out_shape=jax.ShapeDtypeStruct((_NPAD, _W), jnp.float32),
    )(bond_p, ang_p, fb, fa)


def _sc_scatter_wide(x, idx_h, zs):
    mesh = plsc.VectorSubcoreMesh(core_axis_name="c", subcore_axis_name="s")

    @functools.partial(
        pl.kernel,
        mesh=mesh,
        out_type=jax.ShapeDtypeStruct((_NPAD, _W), jnp.float32),
        compiler_params=pltpu.CompilerParams(use_tc_tiling_on_sc=False),
        scratch_types=[
            pltpu.VMEM((128, 128), jnp.float32),
            pltpu.VMEM((_NEIGH, 6, 128), jnp.int32),
            pltpu.VMEM_SHARED((_NPAD, 128), jnp.float32),
        ],
    )
    def k(x_hbm, idx_hbm, zs_hbm, out_hbm, x_v, idx_v, acc):
        cid = lax.axis_index("c")
        sid = lax.axis_index("s")
        zbase = sid * _ZROWS
        pltpu.sync_copy(idx_hbm.at[sid], idx_v)
        for pp in range(_NPAIR):
            pan = pp * 2 + cid
            colc = pan * 128

            @pl.when(pan < _NPAN)
            def _(colc=colc):
                pltpu.sync_copy(zs_hbm, x_v)
                for z in range(_ZROWS // 128):
                    pltpu.sync_copy(x_v, acc.at[pl.ds(zbase + z * 128, 128)])
                plsc.subcore_barrier()
                for j in range(6):
                    pltpu.sync_copy(
                        x_hbm.at[pl.ds(zbase + j * 128, 128),
                                 pl.ds(colc, 128)], x_v)

                    def body(kk, carry, _j=j):
                        pltpu.sync_copy(x_v, acc.at[idx_v.at[kk, _j]],
                                        add=True)
                        return carry

                    lax.fori_loop(0, _NEIGH, body, 0)
                plsc.subcore_barrier()
                for z in range(_ZROWS // 128):
                    pltpu.sync_copy(acc.at[pl.ds(zbase + z * 128, 128)], x_v)
                    pltpu.sync_copy(
                        x_v, out_hbm.at[pl.ds(zbase + z * 128, 128),
                                        pl.ds(colc, 128)])
                plsc.subcore_barrier()

    return k(x, idx_h, zs)


def _sc_scatter64(y, idx_g, zs):
    mesh = plsc.VectorSubcoreMesh(core_axis_name="c", subcore_axis_name="s")

    @functools.partial(
        pl.kernel,
        mesh=mesh,
        out_type=jax.ShapeDtypeStruct((2 * _NPAD, 64), jnp.float32),
        compiler_params=pltpu.CompilerParams(use_tc_tiling_on_sc=False),
        scratch_types=[
            pltpu.VMEM((_CHUNK, 64), jnp.float32),
            pltpu.VMEM((_NEIGH, _NSUB, 128), jnp.int32),
            pltpu.VMEM((128, 64), jnp.float32),
            pltpu.VMEM_SHARED((_NPAD, 64), jnp.float32),
        ],
    )
    def k(y_hbm, idx_hbm, zs_hbm, out_hbm, y_v, idx_v, st_v, acc):
        cid = lax.axis_index("c")
        sid = lax.axis_index("s")
        wid = sid * 2 + cid
        base = wid * _CHUNK
        zbase = sid * _ZROWS
        pltpu.sync_copy(zs_hbm, st_v)
        for z in range(_ZROWS // 128):
            pltpu.sync_copy(st_v, acc.at[pl.ds(zbase + z * 128, 128)])
        pltpu.sync_copy(y_hbm.at[pl.ds(base, _CHUNK)], y_v)
        pltpu.sync_copy(idx_hbm.at[wid], idx_v)
        plsc.subcore_barrier()

        def body(kk, carry):
            for sub in range(_NSUB):
                pltpu.sync_copy(y_v.at[pl.ds(sub * 128, 128)],
                                acc.at[idx_v.at[kk, sub]], add=True)
            return carry

        lax.fori_loop(0, _NEIGH, body, 0)
        plsc.subcore_barrier()
        for z in range(_ZROWS // 128):
            pltpu.sync_copy(acc.at[pl.ds(zbase + z * 128, 128)], st_v)
            pltpu.sync_copy(
                st_v, out_hbm.at[pl.ds(cid * _NPAD + zbase + z * 128, 128)])

    return k(y, idx_g, zs).reshape(2, _NPAD, 64)


def _idx_body(nb_ref, h_ref, g_ref):
    t = jnp.transpose(nb_ref[...], (1, 0))
    h_ref[...] = t.reshape(1, _NEIGH, 6, 128)
    g_ref[...] = t.reshape(_NEIGH, 2, 384).transpose(1, 0, 2).reshape(
        2, _NEIGH, _NSUB, 128)


def _idx_layouts(nbr_p):
    return pl.pallas_call(
        _idx_body,
        grid=(16,),
        in_specs=[pl.BlockSpec((_ZROWS, _NEIGH), lambda i: (i, 0))],
        out_specs=[
            pl.BlockSpec((1, _NEIGH, 6, 128), lambda i: (i, 0, 0, 0)),
            pl.BlockSpec((2, _NEIGH, _NSUB, 128), lambda i: (i, 0, 0, 0)),
        ],
        out_shape=[
            jax.ShapeDtypeStruct((16, _NEIGH, 6, 128), jnp.int32),
            jax.ShapeDtypeStruct((32, _NEIGH, _NSUB, 128), jnp.int32),
        ],
    )(nbr_p)


def _tail1_body(x_ref, agg_ref, w1_ref, w2_ref, bias_ref, x1_ref, st_ref):
    pid = pl.program_id(0)
    h = x_ref[...] + agg_ref[...]
    y = _dot(h, w1_ref[...]) + bias_ref[0][None, :]
    h1 = jnp.maximum(y, 0.0)
    x1 = jnp.maximum(_dot(h1, w2_ref[...]) + bias_ref[1][None, :], 0.0)
    rows = pid * _R + lax.broadcasted_iota(jnp.int32, (_R, 1), 0)
    x1 = jnp.where(rows < _N, x1, 0.0)
    x1_ref[...] = x1

    @pl.when(pid == 0)
    def _():
        st_ref[...] = jnp.zeros_like(st_ref)

    st_ref[...] += jnp.concatenate(
        [jnp.sum(x1, axis=0, keepdims=True),
         jnp.sum(x1 * x1, axis=0, keepdims=True),
         jnp.zeros((6, 64), jnp.float32)], axis=0)


def _tail1(x, agg, w1, w2, bias):
    return pl.pallas_call(
        _tail1_body,
        grid=(_NBLK,),
        in_specs=[
            pl.BlockSpec((_R, _W), lambda i: (i, 0)),
            pl.BlockSpec((_R, _W), lambda i: (i, 0)),
            pl.BlockSpec((_W, 64), lambda i: (0, 0)),
            pl.BlockSpec((64, 64), lambda i: (0, 0)),
            pl.BlockSpec((8, 64), lambda i: (0, 0)),
        ],
        out_specs=[
            pl.BlockSpec((_R, 64), lambda i: (i, 0)),
            pl.BlockSpec((8, 64), lambda i: (0, 0)),
        ],
        out_shape=[
            jax.ShapeDtypeStruct((_NPAD, 64), jnp.float32),
            jax.ShapeDtypeStruct((8, 64), jnp.float32),
        ],
    )(x, agg, w1, w2, bias)


def _bn_body(x_ref, sc_ref, z_ref):
    pid = pl.program_id(0)
    z = ((x_ref[...] - sc_ref[0][None, :]) * sc_ref[1][None, :]
         * sc_ref[2][None, :] + sc_ref[3][None, :])
    rows = pid * _R + lax.broadcasted_iota(jnp.int32, (_R, 1), 0)
    z_ref[...] = jnp.where(rows < _N, z, 0.0)


def _bn(x, sc):
    return pl.pallas_call(
        _bn_body,
        grid=(_NBLK,),
        in_specs=[
            pl.BlockSpec((_R, 64), lambda i: (i, 0)),
            pl.BlockSpec((8, 64), lambda i: (0, 0)),
        ],
        out_specs=pl.BlockSpec((_R, 64), lambda i: (i, 0)),
        out_shape=jax.ShapeDtypeStruct((_NPAD, 64), jnp.float32),
    )(x, sc)


def _tail2_body(z_ref, agg_ref, w1_ref, w2_ref, bias_ref, x2_ref, st_ref):
    pid = pl.program_id(0)
    a = agg_ref[0] + agg_ref[1]
    h = z_ref[...] + a
    y = _dot(h, w1_ref[...]) + bias_ref[0][None, :]
    h1 = jnp.maximum(y, 0.0)
    x2 = jnp.maximum(_dot(h1, w2_ref[...]) + bias_ref[1][None, :], 0.0)
    rows = pid * _R + lax.broadcasted_iota(jnp.int32, (_R, 1), 0)
    x2 = jnp.where(rows < _N, x2, 0.0)
    x2_ref[...] = x2

    @pl.when(pid == 0)
    def _():
        st_ref[...] = jnp.zeros_like(st_ref)

    st_ref[...] += jnp.concatenate(
        [jnp.sum(x2, axis=0, keepdims=True),
         jnp.sum(x2 * x2, axis=0, keepdims=True),
         jnp.zeros((6, 64), jnp.float32)], axis=0)


def _tail2(z, agg, w1, w2, bias):
    return pl.pallas_call(
        _tail2_body,
        grid=(_NBLK,),
        in_specs=[
            pl.BlockSpec((_R, 64), lambda i: (i, 0)),
            pl.BlockSpec((2, _R, 64), lambda i: (0, i, 0)),
            pl.BlockSpec((64, 64), lambda i: (0, 0)),
            pl.BlockSpec((64, 64), lambda i: (0, 0)),
            pl.BlockSpec((8, 64), lambda i: (0, 0)),
        ],
        out_specs=[
            pl.BlockSpec((_R, 64), lambda i: (i, 0)),
            pl.BlockSpec((8, 64), lambda i: (0, 0)),
        ],
        out_shape=[
            jax.ShapeDtypeStruct((_NPAD, 64), jnp.float32),
            jax.ShapeDtypeStruct((8, 64), jnp.float32),
        ],
    )(z, agg, w1, w2, bias)


def _final_body(x_ref, sc_ref, f1_ref, f2_ref, fw_ref, fb_ref, o_ref):
    z = ((x_ref[...] - sc_ref[0][None, :]) * sc_ref[1][None, :]
         * sc_ref[2][None, :] + sc_ref[3][None, :])
    c1 = _dot(z, f1_ref[...]) + sc_ref[4][None, :32]
    c2 = _dot(c1, f2_ref[...]) + sc_ref[5][None, :16]
    o_ref[...] = _dot(c2, fw_ref[...]) + fb_ref[...]


def _final(x, sc, f1, f2, fw, fb):
    return pl.pallas_call(
        _final_body,
        grid=(_NBLK,),
        in_specs=[
            pl.BlockSpec((_R, 64), lambda i: (i, 0)),
            pl.BlockSpec((8, 64), lambda i: (0, 0)),
            pl.BlockSpec((64, 32), lambda i: (0, 0)),
            pl.BlockSpec((32, 16), lambda i: (0, 0)),
            pl.BlockSpec((16, 2), lambda i: (0, 0)),
            pl.BlockSpec((1, 2), lambda i: (0, 0)),
        ],
        out_specs=pl.BlockSpec((_R, 2), lambda i: (i, 0)),
        out_shape=jax.ShapeDtypeStruct((_NPAD, 2), jnp.float32),
    )(x, sc, f1, f2, fw, fb)


def _stats_to_scale(st, g, b):
    mu = st[0] / _N
    var = st[1] / _N - mu * mu
    inv = 1.0 / jnp.sqrt(var + 1e-5)
    return mu, inv, g, b


def _blockdiag(wb, wa, n1, n2):
    return (jnp.zeros((2 * n1, 2 * n2), jnp.float32)
            .at[:n1, :n2].set(wb).at[n1:, n2:].set(wa))


def kernel(bond_fea, angle_fea, species, nbr_idx, crys_idx, params):
    p = params
    f32 = jnp.float32
    bond_p = jnp.zeros((_NPAD, 12), f32).at[:_N].set(bond_fea)
    ang_p = jnp.zeros((_NPAD, 144), f32).at[:_N].set(
        angle_fea.reshape(_N, 144))
    nbr_p = jnp.zeros((_NPAD, _NEIGH), jnp.int32).at[:_N].set(
        nbr_idx.astype(jnp.int32))
    idx_h, idx_g = _idx_layouts(nbr_p)
    zs128 = jnp.zeros((128, 128), f32)
    zs64 = jnp.zeros((128, 64), f32)
    fb = jnp.zeros((8, 64), f32).at[0].set(jnp.linspace(0.0, 8.0, 64))
    fa = jnp.zeros((8, 4), f32).at[0].set(jnp.linspace(-1.0, 1.0, 4))

    x = _k1(bond_p, ang_p, fb, fa)
    agg = _sc_scatter_wide(x, idx_h, zs128)

    w1a_perm = (p['nn1a_W1'].reshape(144, 4, 32)
                .transpose(1, 0, 2).reshape(576, 32))
    w1cat = (jnp.zeros((_W, 64), f32)
             .at[:768, :32].set(p['nn1b_W1'])
             .at[768:1344, 32:].set(w1a_perm)).astype(_BF)
    w2bd = _blockdiag(p['nn1b_W2'], p['nn1a_W2'], 32, 32).astype(_BF)
    bias1 = (jnp.zeros((8, 64), f32)
             .at[0].set(jnp.concatenate([p['nn1b_b1'], p['nn1a_b1']]))
             .at[1].set(jnp.concatenate([p['nn1b_b2'], p['nn1a_b2']])))
    x1, st1 = _tail1(x, agg, w1cat, w2bd, bias1)

    mu1, inv1, g1, b1 = _stats_to_scale(
        st1, jnp.concatenate([p['bn1b_g'], p['bn1a_g']]),
        jnp.concatenate([p['bn1b_b'], p['bn1a_b']]))
    sc1 = (jnp.zeros((8, 64), f32).at[0].set(mu1).at[1].set(inv1)
           .at[2].set(g1).at[3].set(b1))
    z1 = _bn(x1, sc1)

    agg2 = _sc_scatter64(z1, idx_g, zs64)

    w1bd2 = _blockdiag(p['nn2b_W1'], p['nn2a_W1'], 32, 32).astype(_BF)
    w2bd2 = _blockdiag(p['nn2b_W2'], p['nn2a_W2'], 32, 32).astype(_BF)
    bias2 = (jnp.zeros((8, 64), f32)
             .at[0].set(jnp.concatenate([p['nn2b_b1'], p['nn2a_b1']]))
             .at[1].set(jnp.concatenate([p['nn2b_b2'], p['nn2a_b2']])))
    x2, st2 = _tail2(z1, agg2, w1bd2, w2bd2, bias2)

    mu2, inv2, g2, b2 = _stats_to_scale(
        st2, jnp.concatenate([p['bn2b_g'], p['bn2a_g']]),
        jnp.concatenate([p['bn2b_b'], p['bn2a_b']]))
    sc2 = (jnp.zeros((8, 64), f32).at[0].set(mu2).at[1].set(inv2)
           .at[2].set(g2).at[3].set(b2)
           .at[4, :32].set(jnp.concatenate([p['fc1b_b'], p['fc1a_b']]))
           .at[5, :16].set(jnp.concatenate([p['fc2b_b'], p['fc2a_b']])))
    f1 = _blockdiag(p['fc1b_W'], p['fc1a_W'], 32, 16).astype(_BF)
    f2 = _blockdiag(p['fc2b_W'], p['fc2a_W'], 16, 8).astype(_BF)
    out = _final(x2, sc2, f1, f2, p['fc_W'].astype(_BF),
                 p['fc_b'][None, :])
    return out[:_N]
